# baseline (speedup 1.0000x reference)
import functools

import jax
import jax.numpy as jnp
from jax import lax
from jax.experimental import pallas as pl
from jax.experimental.pallas import tpu as pltpu
from jax.experimental.pallas import tpu_sc as plsc

N = 10000
E = 320000
D = 128
BN_EPS = 1e-5

NC = 2
NS = 16
NW = NC * NS
EPW = E // NW
IRW = 80
PADW = IRW * 128 - EPW
NTRASH = 8
CHUNK = 64
NG = 5
GR = IRW // NG
GC = 2 * GR
NBUF = 4
STRIPE = 624
ZCH = 48


def _sc_agg_body(idx_hbm, h_hbm, out_hbm,
                 srcA, srcB, dstA, dstB, rows0, rows1, rows2, rows3,
                 agg_sh, semsA, semsB, semdA, semdB,
                 sem0, sem1, sem2, sem3):
    c = lax.axis_index("c")
    s = lax.axis_index("s")
    wid = s * NC + c
    last = s == NS - 1
    sbufs = [(srcA, semsA), (srcB, semsB)]
    dbufs = [(dstA, semdA), (dstB, semdB)]
    rows = [(rows0, sem0), (rows1, sem1), (rows2, sem2), (rows3, sem3)]

    def _idx_copy(i, g, buf, sem):
        return pltpu.async_copy(idx_hbm.at[i, wid, pl.ds(g * GR, GR)], buf, sem)

    def _idx_wait(i, g, buf, sem):
        pltpu.make_async_copy(idx_hbm.at[i, wid, pl.ds(g * GR, GR)], buf,
                              sem).wait()

    _idx_copy(0, 0, srcA, semsA)
    _idx_copy(1, 0, dstA, semdA)
    _idx_copy(0, 1, srcB, semsB)
    _idx_copy(1, 1, dstB, semdB)

    def _zrow(r, carry):
        for k in range(D // 16):
            rows0[r, pl.ds(k * 16, 16)] = jnp.zeros((16,), jnp.float32)
        return carry
    lax.fori_loop(0, ZCH, _zrow, 0)
    zcps = [pltpu.async_copy(rows0.at[pl.ds(0, ZCH)],
                             agg_sh.at[pl.ds(s * STRIPE + z * ZCH, ZCH)], sem0)
            for z in range(STRIPE // ZCH)]

    @pl.when(last)
    def _():
        pltpu.async_copy(rows0.at[pl.ds(0, N + NTRASH - NS * STRIPE)],
                         agg_sh.at[pl.ds(NS * STRIPE, N + NTRASH - NS * STRIPE)],
                         sem1).wait()
    for cp in zcps:
        cp.wait()
    plsc.subcore_barrier()

    def _sub(iv, j_half, j_row):
        return iv.at[j_row, pl.ds(j_half * CHUNK, CHUNK)]

    def _gather(iv, j_half, j_row, buf, sem):
        return pltpu.async_copy(h_hbm.at[_sub(iv, j_half, j_row)], buf, sem)

    def _wait(iv, j_half, j_row, buf, sem):
        pltpu.make_async_copy(h_hbm.at[_sub(iv, j_half, j_row)], buf,
                              sem).wait()

    _idx_wait(0, 0, srcA, semsA)
    _idx_wait(1, 0, dstA, semdA)
    for b in range(NBUF):
        _gather(srcA, b % 2, b // 2, rows[b][0], rows[b][1])

    for g in range(NG):
        src_v, _ = sbufs[g % 2]
        dst_v, _ = dbufs[g % 2]

        def _block(t, carry):
            for b in range(NBUF):
                r = t * (NBUF // 2) + b // 2
                h2 = b % 2
                _wait(src_v, h2, r, rows[b][0], rows[b][1])
                pltpu.sync_copy(rows[b][0], agg_sh.at[_sub(dst_v, h2, r)],
                                add=True)
                _gather(src_v, h2, r + NBUF // 2, rows[b][0], rows[b][1])
            return carry
        lax.fori_loop(0, GC // NBUF - 1, _block, 0)

        tail_r0 = (GC - NBUF) // 2
        if g + 1 < NG:
            nsrc, nssem = sbufs[(g + 1) % 2]
            ndst, ndsem = dbufs[(g + 1) % 2]
            _idx_wait(0, g + 1, nsrc, nssem)
            _idx_wait(1, g + 1, ndst, ndsem)
            for b in range(NBUF):
                r = tail_r0 + b // 2
                h2 = b % 2
                _wait(src_v, h2, r, rows[b][0], rows[b][1])
                pltpu.sync_copy(rows[b][0], agg_sh.at[_sub(dst_v, h2, r)],
                                add=True)
                _gather(nsrc, h2, b // 2, rows[b][0], rows[b][1])
            if g + 2 < NG:
                _idx_copy(0, g + 2, src_v, sbufs[g % 2][1])
                _idx_copy(1, g + 2, dst_v, dbufs[g % 2][1])
        else:
            for b in range(NBUF):
                r = tail_r0 + b // 2
                h2 = b % 2
                _wait(src_v, h2, r, rows[b][0], rows[b][1])
                pltpu.sync_copy(rows[b][0], agg_sh.at[_sub(dst_v, h2, r)],
                                add=True)
    plsc.subcore_barrier()

    pltpu.sync_copy(agg_sh.at[pl.ds(s * STRIPE, STRIPE)],
                    out_hbm.at[c, pl.ds(s * STRIPE, STRIPE)])

    @pl.when(last)
    def _():
        pltpu.sync_copy(agg_sh.at[pl.ds(NS * STRIPE, N - NS * STRIPE)],
                        out_hbm.at[c, pl.ds(NS * STRIPE, N - NS * STRIPE)])


def _make_sc_agg():
    return functools.partial(
        pl.kernel,
        out_type=jax.ShapeDtypeStruct((NC, N, D), jnp.float32),
        mesh=plsc.VectorSubcoreMesh(core_axis_name="c", subcore_axis_name="s",
                                    num_cores=NC, num_subcores=NS),
        scratch_types=[
            pltpu.VMEM((GR, 128), jnp.int32),
            pltpu.VMEM((GR, 128), jnp.int32),
            pltpu.VMEM((GR, 128), jnp.int32),
            pltpu.VMEM((GR, 128), jnp.int32),
            pltpu.VMEM((CHUNK, D), jnp.float32),
            pltpu.VMEM((CHUNK, D), jnp.float32),
            pltpu.VMEM((CHUNK, D), jnp.float32),
            pltpu.VMEM((CHUNK, D), jnp.float32),
            pltpu.VMEM_SHARED((N + NTRASH, D), jnp.float32),
            pltpu.SemaphoreType.DMA,
            pltpu.SemaphoreType.DMA,
            pltpu.SemaphoreType.DMA,
            pltpu.SemaphoreType.DMA,
            pltpu.SemaphoreType.DMA,
            pltpu.SemaphoreType.DMA,
            pltpu.SemaphoreType.DMA,
            pltpu.SemaphoreType.DMA,
        ],
    )(_sc_agg_body)


def _tc_body(h_ref, p_ref, eps_ref, W1_ref, b1_ref, W2_ref, b2_ref,
             g_ref, bt_ref, o_ref):
    x = h_ref[...] * (1.0 + eps_ref[0]) + p_ref[0] + p_ref[1]
    x = jnp.dot(x, W1_ref[...], preferred_element_type=jnp.float32)
    x = jnp.maximum(x + b1_ref[...], 0.0)
    x = jnp.dot(x, W2_ref[...], preferred_element_type=jnp.float32)
    x = x + b2_ref[...]
    mean = jnp.mean(x, axis=0, keepdims=True)
    xc = x - mean
    var = jnp.mean(xc * xc, axis=0, keepdims=True)
    y = xc * lax.rsqrt(var + BN_EPS) * g_ref[...] + bt_ref[...]
    o_ref[...] = jnp.maximum(y, 0.0)


def kernel(h, edge_index, eps, W1, b1, W2, b2, gamma, beta):
    eb = edge_index.astype(jnp.int32).reshape(2, NW, EPW)
    lane = jnp.arange(NW * PADW, dtype=jnp.int32).reshape(NW, PADW)
    src_pad = lane * 41 % N
    dst_pad = lane % NTRASH + N
    eidx = jnp.stack([jnp.concatenate([eb[0], src_pad], axis=1),
                      jnp.concatenate([eb[1], dst_pad], axis=1)])
    eidx = eidx.reshape(2, NW, IRW, 128)
    partials = _make_sc_agg()(eidx, h)
    vspec = pl.BlockSpec(memory_space=pltpu.VMEM)
    out = pl.pallas_call(
        _tc_body,
        out_shape=jax.ShapeDtypeStruct((N, D), jnp.float32),
        in_specs=[vspec, vspec, pl.BlockSpec(memory_space=pltpu.SMEM),
                  vspec, vspec, vspec, vspec, vspec, vspec],
        out_specs=vspec,
    )(h, partials, eps, W1, b1.reshape(1, D), W2, b2.reshape(1, D),
      gamma.reshape(1, D), beta.reshape(1, D))
    return out


# speedup vs baseline: 1.1420x; 1.1420x over previous
import functools

import jax
import jax.numpy as jnp
from jax import lax
from jax.experimental import pallas as pl
from jax.experimental.pallas import tpu as pltpu
from jax.experimental.pallas import tpu_sc as plsc

N = 10000
E = 320000
D = 128
BN_EPS = 1e-5

NC = 2
NS = 16
NW = NC * NS
EPW = E // NW
BPW = 78
CHUNK = 64
NG = 13
GB = BPW // NG
GE = GB * 128
GC = 2 * GB
NBUF = 4
STRIPE = 624
ZCH = 48


def _sc_agg_body(idx_hbm, h_hbm, out_hbm,
                 ibufA, ibufB, ebuf, rows0, rows1, rows2, rows3,
                 agg_sh, semiA, semiB, sem0, sem1, sem2, sem3):
    c = lax.axis_index("c")
    s = lax.axis_index("s")
    wid = s * NC + c
    last = s == NS - 1
    ibufs = [(ibufA, semiA), (ibufB, semiB)]
    rows = [(rows0, sem0), (rows1, sem1), (rows2, sem2), (rows3, sem3)]
    base_e = wid * (BPW * 128)

    def _idx_copy(g, buf, sem):
        return pltpu.async_copy(
            idx_hbm.at[:, pl.ds(base_e + g * GE, GE)], buf, sem)

    def _idx_wait(g, buf, sem):
        pltpu.make_async_copy(
            idx_hbm.at[:, pl.ds(base_e + g * GE, GE)], buf, sem).wait()

    _idx_copy(0, ibufA, semiA)
    _idx_copy(1, ibufB, semiB)

    def _zrow(r, carry):
        for k in range(D // 16):
            rows0[r, pl.ds(k * 16, 16)] = jnp.zeros((16,), jnp.float32)
        return carry
    lax.fori_loop(0, ZCH, _zrow, 0)
    zcps = [pltpu.async_copy(rows0.at[pl.ds(0, ZCH)],
                             agg_sh.at[pl.ds(s * STRIPE + z * ZCH, ZCH)], sem0)
            for z in range(STRIPE // ZCH)]

    @pl.when(last)
    def _():
        pltpu.async_copy(rows0.at[pl.ds(0, 16)],
                         agg_sh.at[pl.ds(NS * STRIPE, N - NS * STRIPE)],
                         sem1).wait()
    for cp in zcps:
        cp.wait()
    plsc.subcore_barrier()

    def _gather(iv, j, buf, sem):
        return pltpu.async_copy(
            h_hbm.at[iv.at[0, pl.ds(j * CHUNK, CHUNK)]], buf, sem)

    def _wait(iv, j, buf, sem):
        pltpu.make_async_copy(
            h_hbm.at[iv.at[0, pl.ds(j * CHUNK, CHUNK)]], buf, sem).wait()

    def _scatter(iv, j, buf):
        pltpu.sync_copy(buf, agg_sh.at[iv.at[1, pl.ds(j * CHUNK, CHUNK)]],
                        add=True)

    _idx_wait(0, ibufA, semiA)
    for b in range(NBUF):
        _gather(ibufA, b, rows[b][0], rows[b][1])

    for g in range(NG):
        idx_v, _ = ibufs[g % 2]

        def _block(t, carry):
            for b in range(NBUF):
                j = t * NBUF + b
                _wait(idx_v, j, rows[b][0], rows[b][1])
                _scatter(idx_v, j, rows[b][0])
                _gather(idx_v, j + NBUF, rows[b][0], rows[b][1])
            return carry
        lax.fori_loop(0, GC // NBUF - 1, _block, 0)

        if g + 1 < NG:
            nidx, nsem = ibufs[(g + 1) % 2]
            _idx_wait(g + 1, nidx, nsem)
            for b in range(NBUF):
                j = GC - NBUF + b
                _wait(idx_v, j, rows[b][0], rows[b][1])
                _scatter(idx_v, j, rows[b][0])
                _gather(nidx, b, rows[b][0], rows[b][1])
            if g + 2 < NG:
                _idx_copy(g + 2, idx_v, ibufs[g % 2][1])
        else:
            for b in range(NBUF):
                j = GC - NBUF + b
                _wait(idx_v, j, rows[b][0], rows[b][1])
                _scatter(idx_v, j, rows[b][0])

    @pl.when(wid < E // 128 - NW * BPW)
    def _():
        off = (NW * BPW + wid) * 128
        pltpu.sync_copy(idx_hbm.at[:, pl.ds(off, 128)], ebuf)
        for h2 in range(2):
            pltpu.async_copy(
                h_hbm.at[ebuf.at[0, pl.ds(h2 * CHUNK, CHUNK)]],
                rows0, sem0).wait()
            pltpu.sync_copy(
                rows0, agg_sh.at[ebuf.at[1, pl.ds(h2 * CHUNK, CHUNK)]],
                add=True)
    plsc.subcore_barrier()

    pltpu.sync_copy(agg_sh.at[pl.ds(s * STRIPE, STRIPE)],
                    out_hbm.at[c, pl.ds(s * STRIPE, STRIPE)])

    @pl.when(last)
    def _():
        pltpu.sync_copy(agg_sh.at[pl.ds(NS * STRIPE, N - NS * STRIPE)],
                        out_hbm.at[c, pl.ds(NS * STRIPE, N - NS * STRIPE)])


def _make_sc_agg():
    return functools.partial(
        pl.kernel,
        out_type=jax.ShapeDtypeStruct((NC, N, D), jnp.float32),
        mesh=plsc.VectorSubcoreMesh(core_axis_name="c", subcore_axis_name="s",
                                    num_cores=NC, num_subcores=NS),
        scratch_types=[
            pltpu.VMEM((2, GE), jnp.int32),
            pltpu.VMEM((2, GE), jnp.int32),
            pltpu.VMEM((2, 128), jnp.int32),
            pltpu.VMEM((CHUNK, D), jnp.float32),
            pltpu.VMEM((CHUNK, D), jnp.float32),
            pltpu.VMEM((CHUNK, D), jnp.float32),
            pltpu.VMEM((CHUNK, D), jnp.float32),
            pltpu.VMEM_SHARED((N, D), jnp.float32),
            pltpu.SemaphoreType.DMA,
            pltpu.SemaphoreType.DMA,
            pltpu.SemaphoreType.DMA,
            pltpu.SemaphoreType.DMA,
            pltpu.SemaphoreType.DMA,
            pltpu.SemaphoreType.DMA,
        ],
    )(_sc_agg_body)


def _tc_body(h_ref, p_ref, eps_ref, W1_ref, b1_ref, W2_ref, b2_ref,
             g_ref, bt_ref, o_ref):
    x = h_ref[...] * (1.0 + eps_ref[0]) + p_ref[0] + p_ref[1]
    x = jnp.dot(x, W1_ref[...], preferred_element_type=jnp.float32)
    x = jnp.maximum(x + b1_ref[...], 0.0)
    x = jnp.dot(x, W2_ref[...], preferred_element_type=jnp.float32)
    x = x + b2_ref[...]
    mean = jnp.mean(x, axis=0, keepdims=True)
    xc = x - mean
    var = jnp.mean(xc * xc, axis=0, keepdims=True)
    y = xc * lax.rsqrt(var + BN_EPS) * g_ref[...] + bt_ref[...]
    o_ref[...] = jnp.maximum(y, 0.0)


def kernel(h, edge_index, eps, W1, b1, W2, b2, gamma, beta):
    partials = _make_sc_agg()(edge_index.astype(jnp.int32), h)
    vspec = pl.BlockSpec(memory_space=pltpu.VMEM)
    out = pl.pallas_call(
        _tc_body,
        out_shape=jax.ShapeDtypeStruct((N, D), jnp.float32),
        in_specs=[vspec, vspec, pl.BlockSpec(memory_space=pltpu.SMEM),
                  vspec, vspec, vspec, vspec, vspec, vspec],
        out_specs=vspec,
    )(h, partials, eps, W1, b1.reshape(1, D), W2, b2.reshape(1, D),
      gamma.reshape(1, D), beta.reshape(1, D))
    return out
